# baseline (speedup 1.0000x reference)
import functools

import jax
import jax.numpy as jnp
from jax import lax
from jax.experimental import pallas as pl
from jax.experimental.pallas import tpu as pltpu
from jax.experimental.pallas import tpu_sc as plsc

N = 10000
E = 160000
D = 256
R = 8
NB = 30
EPS = 1e-5

NC = 2
NS = 16
NHALF = 5120
NPAD = NC * NHALF
CNTN = NHALF * R
EPAD = 163840
EPT = EPAD // NS
CB = 128
NCHC = EPT // CB
RB = 64
NCHR = EPT // RB
CSL = CNTN // NS
ROWS_T = NHALF // NS
KMASK = 131071

_f32 = jnp.float32
_i32 = jnp.int32


def _emb_gather(emb, xpad):
    bpw = NPAD // (NC * NS)
    mesh = plsc.VectorSubcoreMesh(core_axis_name="c", subcore_axis_name="s")

    @functools.partial(
        pl.kernel, mesh=mesh,
        out_type=jax.ShapeDtypeStruct((NPAD, D), _f32),
        scratch_types=[
            pltpu.VMEM((bpw,), _i32),
            pltpu.VMEM((bpw, D), _f32),
            pltpu.SemaphoreType.DMA,
        ],
    )
    def k(emb_hbm, idx_hbm, out_hbm, idx_v, rows_v, sem):
        wid = lax.axis_index("s") * NC + lax.axis_index("c")
        base = wid * bpw
        pltpu.sync_copy(idx_hbm.at[pl.ds(base, bpw)], idx_v)
        pltpu.async_copy(emb_hbm.at[idx_v], rows_v, sem).wait()
        pltpu.sync_copy(rows_v, out_hbm.at[pl.ds(base, bpw)])

    return k(emb, xpad)


def _wcat(comp, basisf):
    BK = 2048

    def body(c_ref, b_ref, o_ref):
        o_ref[...] = jnp.dot(c_ref[...], b_ref[...],
                             preferred_element_type=_f32)

    return pl.pallas_call(
        body,
        grid=(D * D // BK,),
        in_specs=[pl.BlockSpec((R, NB), lambda j: (0, 0)),
                  pl.BlockSpec((NB, BK), lambda j: (0, j))],
        out_specs=pl.BlockSpec((R, BK), lambda j: (0, j)),
        out_shape=jax.ShapeDtypeStruct((R, D * D), _f32),
    )(comp, basisf)


def _xw(h, w3):
    BM = 256
    gm = pl.cdiv(N, BM)

    def body(h_ref, w_ref, o_ref):
        o_ref[...] = jnp.dot(h_ref[...], w_ref[0],
                             preferred_element_type=_f32)

    return pl.pallas_call(
        body,
        grid=(gm, R),
        in_specs=[pl.BlockSpec((BM, D), lambda i, r: (i, 0)),
                  pl.BlockSpec((1, D, D), lambda i, r: (r, 0, 0))],
        out_specs=pl.BlockSpec((BM, D), lambda i, r: (i, r)),
        out_shape=jax.ShapeDtypeStruct((N, R * D), _f32),
    )(h, w3)


TSL = CNTN // NS
TROWS = NHALF // NS
SCB = 2048
NSC1 = EPT // SCB
NSC3 = EPAD // SCB
GB = 64


def _edge_agg(xw2, ewf):
    mesh = plsc.VectorSubcoreMesh(core_axis_name="c", subcore_axis_name="s")

    @functools.partial(
        pl.kernel, mesh=mesh,
        compiler_params=pltpu.CompilerParams(needs_layout_passes=False),
        out_type=jax.ShapeDtypeStruct((NPAD * D,), _f32),
        scratch_types=[
            pltpu.VMEM((SCB,), _i32),
            pltpu.VMEM((SCB,), _i32),
            pltpu.VMEM((SCB + 16,), _i32),
            pltpu.VMEM((NS, CB), _i32),
            pltpu.VMEM((NS, CB), _f32),
            pltpu.VMEM((GB,), _i32),
            pltpu.VMEM((GB,), _i32),
            pltpu.VMEM((GB,), _f32),
            pltpu.VMEM((GB, D), _f32),
            pltpu.VMEM((TSL,), _f32),
            pltpu.VMEM((TROWS * D,), _f32),
            pltpu.VMEM_SHARED((CNTN,), _f32),
            pltpu.SemaphoreType.DMA,
            pltpu.SemaphoreType.DMA,
        ],
    )
    def k(xw_hbm, ew_hbm, agg_hbm,
          ew_a, ew_b, cew_v, cidx_v, cval_v, gidx_v, didx_v, val_v, rows_v,
          norm_v, acc_v, cnt_sh, sem, sem2):
        c = lax.axis_index("c")
        s = lax.axis_index("s")
        klo = c * CNTN
        wlo = klo + s * TSL

        def zl(i, _):
            norm_v[pl.ds(i * 16, 16)] = jnp.zeros((16,), _f32)
            return 0
        lax.fori_loop(0, TSL // 16, zl, 0)
        pltpu.sync_copy(norm_v, cnt_sh.at[pl.ds(s * TSL, TSL)])
        def zr(i):
            acc_v[pl.ds(i * 16, 16)] = jnp.zeros((16,), _f32)
        plsc.parallel_loop(0, TROWS * D // 16, unroll=8)(zr)
        plsc.subcore_barrier()

        def cnt_chunk(ci, _):
            pltpu.sync_copy(
                ew_hbm.at[pl.ds(s * EPT + ci * SCB, SCB)], ew_a)
            def sub(si, _):
                def grp(gi):
                    ew = ew_a[pl.ds(si * CB + gi * 16, 16)]
                    key = ew & KMASK
                    member = (key >= klo) & (key < klo + CNTN)
                    lk = jnp.where(key >= CNTN, key - CNTN, key)
                    cidx_v[si, pl.ds(gi * 16, 16)] = lk
                    cval_v[si, pl.ds(gi * 16, 16)] = jnp.where(
                        member, 1.0, 0.0)
                plsc.parallel_loop(0, CB // 16, unroll=4)(grp)
                pltpu.async_copy(cval_v.at[si], cnt_sh.at[cidx_v.at[si]],
                                 sem2, add=True)
                return 0
            lax.fori_loop(0, SCB // CB, sub, 0)
            def drain(si, _):
                pltpu.make_async_copy(cval_v.at[si],
                                      cnt_sh.at[cidx_v.at[si]], sem2).wait()
                return 0
            lax.fori_loop(0, SCB // CB, drain, 0)
            return 0
        lax.fori_loop(0, NSC1, cnt_chunk, 0)
        plsc.subcore_barrier()

        pltpu.sync_copy(cnt_sh.at[pl.ds(s * TSL, TSL)], norm_v)
        def nl(i, _):
            v = norm_v[pl.ds(i * 16, 16)]
            norm_v[pl.ds(i * 16, 16)] = 1.0 / jnp.maximum(v, 1.0)
            return 0
        lax.fori_loop(0, TSL // 16, nl, 0)

        def _fire(slot_ref, slot_sem, idx):
            pltpu.async_copy(ew_hbm.at[pl.ds(idx * SCB, SCB)], slot_ref,
                             slot_sem)

        def _wait(slot_ref, slot_sem, idx):
            pltpu.make_async_copy(ew_hbm.at[pl.ds(idx * SCB, SCB)], slot_ref,
                                  slot_sem).wait()

        def _process(slot_ref):
            def scan(g, off):
                ew = slot_ref[pl.ds(g * 16, 16)]
                key = ew & KMASK
                m = (key >= wlo) & (key < wlo + TSL)
                plsc.store_compressed(cew_v.at[pl.ds(off, 16)], ew, mask=m)
                return off + plsc.all_reduce_population_count(m)[0]
            off = plsc.parallel_loop(0, SCB // 16, unroll=4,
                                     carry=jnp.int32(0))(scan)
            nsub = (off + GB - 1) // GB
            def sub(j, _):
                base2 = j * GB
                def grp(gi):
                    cw = cew_v[pl.ds(base2 + gi * 16, 16)]
                    key = cw & KMASK
                    nloc = jnp.clip(key - wlo, 0, TSL - 1)
                    gidx_v[pl.ds(gi * 16, 16)] = jnp.clip(
                        ((cw >> 17) << 3) | (key & 7), 0, N * R - 1)
                    didx_v[pl.ds(gi * 16, 16)] = nloc >> 3
                    live = (base2 + gi * 16 + lax.iota(_i32, 16)) < off
                    nv = plsc.load_gather(norm_v, [nloc])
                    val_v[pl.ds(gi * 16, 16)] = jnp.where(live, nv, 0.0)
                plsc.parallel_loop(0, GB // 16, unroll=2)(grp)
                pltpu.async_copy(xw_hbm.at[gidx_v], rows_v, sem).wait()
                def acc(gi, _):
                    dl16 = didx_v[pl.ds(gi * 16, 16)]
                    sv16 = val_v[pl.ds(gi * 16, 16)]
                    for l in range(16):
                        base = dl16[l] * D
                        sv = sv16[l]
                        j = gi * 16 + l
                        for kk in range(D // 16):
                            acc_v[pl.ds(base + kk * 16, 16)] = (
                                acc_v[pl.ds(base + kk * 16, 16)]
                                + rows_v[j, pl.ds(kk * 16, 16)] * sv)
                    return 0
                lax.fori_loop(0, GB // 16, acc, 0)
                return 0
            lax.fori_loop(0, nsub, sub, 0)

        _fire(ew_a, sem, 0)
        _fire(ew_b, sem2, 1)
        def chunk2(c2, _):
            i0 = c2 * 2
            _wait(ew_a, sem, i0)
            _process(ew_a)
            @pl.when(i0 + 2 < NSC3)
            def _():
                _fire(ew_a, sem, i0 + 2)
            _wait(ew_b, sem2, i0 + 1)
            _process(ew_b)
            @pl.when(i0 + 3 < NSC3)
            def _():
                _fire(ew_b, sem2, i0 + 3)
            return 0
        lax.fori_loop(0, NSC3 // 2, chunk2, 0)

        pltpu.sync_copy(
            acc_v,
            agg_hbm.at[pl.ds((c * NHALF + s * TROWS) * D, TROWS * D)])

    return k(xw2, ewf).reshape(NPAD, D)


def _post(agg, h, root, bias, g, b):
    BM = 256
    gm = pl.cdiv(N, BM)

    def body(a_ref, h_ref, r_ref, bias_ref, g_ref, b_ref, o_ref):
        y = a_ref[...] + jnp.dot(h_ref[...], r_ref[...],
                                 preferred_element_type=_f32) + bias_ref[...]
        y = jnp.maximum(y, 0.0)
        mu = jnp.mean(y, axis=-1, keepdims=True)
        var = jnp.mean((y - mu) ** 2, axis=-1, keepdims=True)
        o_ref[...] = (y - mu) * lax.rsqrt(var + EPS) * g_ref[...] + b_ref[...]

    return pl.pallas_call(
        body,
        grid=(gm,),
        in_specs=[pl.BlockSpec((BM, D), lambda i: (i, 0)),
                  pl.BlockSpec((BM, D), lambda i: (i, 0)),
                  pl.BlockSpec((D, D), lambda i: (0, 0)),
                  pl.BlockSpec((1, D), lambda i: (0, 0)),
                  pl.BlockSpec((1, D), lambda i: (0, 0)),
                  pl.BlockSpec((1, D), lambda i: (0, 0))],
        out_specs=pl.BlockSpec((BM, D), lambda i: (i, 0)),
        out_shape=jax.ShapeDtypeStruct((N, D), _f32),
    )(agg, h, root, bias, g, b)


def _layer(h, ew3, basis, comp, root, bias, g, b):
    w3 = _wcat(comp, basis.reshape(NB, D * D)).reshape(R, D, D)
    xw = _xw(h, w3)
    agg = _edge_agg(xw.reshape(N * R, D), ew3)
    return _post(agg[:N], h, root, bias.reshape(1, D), g.reshape(1, D),
                 b.reshape(1, D))


def kernel(x, edge_index, edge_type, emb, basis1, comp1, root1, bias1, g1, b1,
           basis2, comp2, root2, bias2, g2, b2):
    xpad = jnp.concatenate(
        [x.astype(_i32), jnp.zeros((NPAD - N,), _i32)])
    srcp = jnp.concatenate(
        [edge_index[0].astype(_i32), jnp.zeros((EPAD - E,), _i32)])
    keyp = jnp.concatenate(
        [edge_index[1].astype(_i32) * R + edge_type.astype(_i32),
         jnp.full((EPAD - E,), (NPAD - 1) * R, _i32)])
    ew3 = (srcp << 17) | keyp

    h0 = _emb_gather(emb, xpad)[:N]
    h1 = _layer(h0, ew3, basis1, comp1, root1, bias1, g1, b1)
    h2 = _layer(h1, ew3, basis2, comp2, root2, bias2, g2, b2)
    return h2


# speedup vs baseline: 1.0183x; 1.0183x over previous
import functools

import jax
import jax.numpy as jnp
from jax import lax
from jax.experimental import pallas as pl
from jax.experimental.pallas import tpu as pltpu
from jax.experimental.pallas import tpu_sc as plsc

N = 10000
E = 160000
D = 256
R = 8
NB = 30
EPS = 1e-5

NC = 2
NS = 16
NHALF = 5120
NPAD = NC * NHALF
CNTN = NHALF * R
EPAD = 163840
EPT = EPAD // NS
CB = 128
NCHC = EPT // CB
RB = 64
NCHR = EPT // RB
CSL = CNTN // NS
ROWS_T = NHALF // NS
KMASK = 131071

_f32 = jnp.float32
_i32 = jnp.int32


def _emb_gather(emb, xpad):
    bpw = NPAD // (NC * NS)
    mesh = plsc.VectorSubcoreMesh(core_axis_name="c", subcore_axis_name="s")

    @functools.partial(
        pl.kernel, mesh=mesh,
        out_type=jax.ShapeDtypeStruct((NPAD, D), _f32),
        scratch_types=[
            pltpu.VMEM((bpw,), _i32),
            pltpu.VMEM((bpw, D), _f32),
            pltpu.SemaphoreType.DMA,
        ],
    )
    def k(emb_hbm, idx_hbm, out_hbm, idx_v, rows_v, sem):
        wid = lax.axis_index("s") * NC + lax.axis_index("c")
        base = wid * bpw
        pltpu.sync_copy(idx_hbm.at[pl.ds(base, bpw)], idx_v)
        pltpu.async_copy(emb_hbm.at[idx_v], rows_v, sem).wait()
        pltpu.sync_copy(rows_v, out_hbm.at[pl.ds(base, bpw)])

    return k(emb, xpad)


def _wcat(comp, basisf):
    BK = 2048

    def body(c_ref, b_ref, o_ref):
        o_ref[...] = jnp.dot(c_ref[...], b_ref[...],
                             preferred_element_type=_f32)

    return pl.pallas_call(
        body,
        grid=(D * D // BK,),
        in_specs=[pl.BlockSpec((R, NB), lambda j: (0, 0)),
                  pl.BlockSpec((NB, BK), lambda j: (0, j))],
        out_specs=pl.BlockSpec((R, BK), lambda j: (0, j)),
        out_shape=jax.ShapeDtypeStruct((R, D * D), _f32),
    )(comp, basisf)


def _xw(h, w3):
    BM = 256
    gm = pl.cdiv(N, BM)

    def body(h_ref, w_ref, o_ref):
        o_ref[...] = jnp.dot(h_ref[...].astype(jnp.bfloat16),
                             w_ref[0].astype(jnp.bfloat16),
                             preferred_element_type=_f32)

    return pl.pallas_call(
        body,
        grid=(gm, R),
        in_specs=[pl.BlockSpec((BM, D), lambda i, r: (i, 0)),
                  pl.BlockSpec((1, D, D), lambda i, r: (r, 0, 0))],
        out_specs=pl.BlockSpec((BM, D), lambda i, r: (i, r)),
        out_shape=jax.ShapeDtypeStruct((N, R * D), _f32),
    )(h, w3)


TSL = CNTN // NS
TROWS = NHALF // NS
SCB = 2048
NSC1 = EPT // SCB
NSC3 = EPAD // SCB
GB = 64


def _edge_agg(xw2, ewf):
    mesh = plsc.VectorSubcoreMesh(core_axis_name="c", subcore_axis_name="s")

    @functools.partial(
        pl.kernel, mesh=mesh,
        compiler_params=pltpu.CompilerParams(needs_layout_passes=False),
        out_type=jax.ShapeDtypeStruct((NPAD * D,), _f32),
        scratch_types=[
            pltpu.VMEM((SCB,), _i32),
            pltpu.VMEM((SCB,), _i32),
            pltpu.VMEM((SCB + 16,), _i32),
            pltpu.VMEM((NS, CB), _i32),
            pltpu.VMEM((NS, CB), _f32),
            pltpu.VMEM((GB,), _i32),
            pltpu.VMEM((GB,), _i32),
            pltpu.VMEM((GB,), _f32),
            pltpu.VMEM((GB, D), _f32),
            pltpu.VMEM((TSL,), _f32),
            pltpu.VMEM((TROWS * D,), _f32),
            pltpu.VMEM_SHARED((CNTN,), _f32),
            pltpu.SemaphoreType.DMA,
            pltpu.SemaphoreType.DMA,
        ],
    )
    def k(xw_hbm, ew_hbm, agg_hbm,
          ew_a, ew_b, cew_v, cidx_v, cval_v, gidx_v, didx_v, val_v, rows_v,
          norm_v, acc_v, cnt_sh, sem, sem2):
        c = lax.axis_index("c")
        s = lax.axis_index("s")
        klo = c * CNTN
        wlo = klo + s * TSL

        def zl(i, _):
            norm_v[pl.ds(i * 16, 16)] = jnp.zeros((16,), _f32)
            return 0
        lax.fori_loop(0, TSL // 16, zl, 0)
        pltpu.sync_copy(norm_v, cnt_sh.at[pl.ds(s * TSL, TSL)])
        def zr(i):
            acc_v[pl.ds(i * 16, 16)] = jnp.zeros((16,), _f32)
        plsc.parallel_loop(0, TROWS * D // 16, unroll=8)(zr)
        plsc.subcore_barrier()

        def cnt_chunk(ci, _):
            pltpu.sync_copy(
                ew_hbm.at[pl.ds(s * EPT + ci * SCB, SCB)], ew_a)
            def sub(si, _):
                def grp(gi):
                    ew = ew_a[pl.ds(si * CB + gi * 16, 16)]
                    key = ew & KMASK
                    member = (key >= klo) & (key < klo + CNTN)
                    lk = jnp.where(key >= CNTN, key - CNTN, key)
                    cidx_v[si, pl.ds(gi * 16, 16)] = lk
                    cval_v[si, pl.ds(gi * 16, 16)] = jnp.where(
                        member, 1.0, 0.0)
                plsc.parallel_loop(0, CB // 16, unroll=4)(grp)
                pltpu.async_copy(cval_v.at[si], cnt_sh.at[cidx_v.at[si]],
                                 sem2, add=True)
                return 0
            lax.fori_loop(0, SCB // CB, sub, 0)
            def drain(si, _):
                pltpu.make_async_copy(cval_v.at[si],
                                      cnt_sh.at[cidx_v.at[si]], sem2).wait()
                return 0
            lax.fori_loop(0, SCB // CB, drain, 0)
            return 0
        lax.fori_loop(0, NSC1, cnt_chunk, 0)
        plsc.subcore_barrier()

        pltpu.sync_copy(cnt_sh.at[pl.ds(s * TSL, TSL)], norm_v)
        def nl(i, _):
            v = norm_v[pl.ds(i * 16, 16)]
            norm_v[pl.ds(i * 16, 16)] = 1.0 / jnp.maximum(v, 1.0)
            return 0
        lax.fori_loop(0, TSL // 16, nl, 0)

        def _fire(slot_ref, slot_sem, idx):
            pltpu.async_copy(ew_hbm.at[pl.ds(idx * SCB, SCB)], slot_ref,
                             slot_sem)

        def _wait(slot_ref, slot_sem, idx):
            pltpu.make_async_copy(ew_hbm.at[pl.ds(idx * SCB, SCB)], slot_ref,
                                  slot_sem).wait()

        def _process(slot_ref):
            def scan(g, off):
                ew = slot_ref[pl.ds(g * 16, 16)]
                key = ew & KMASK
                m = (key >= wlo) & (key < wlo + TSL)
                plsc.store_compressed(cew_v.at[pl.ds(off, 16)], ew, mask=m)
                return off + plsc.all_reduce_population_count(m)[0]
            off = plsc.parallel_loop(0, SCB // 16, unroll=4,
                                     carry=jnp.int32(0))(scan)
            nsub = (off + GB - 1) // GB
            def sub(j, _):
                base2 = j * GB
                def grp(gi):
                    cw = cew_v[pl.ds(base2 + gi * 16, 16)]
                    key = cw & KMASK
                    nloc = jnp.clip(key - wlo, 0, TSL - 1)
                    gidx_v[pl.ds(gi * 16, 16)] = jnp.clip(
                        ((cw >> 17) << 3) | (key & 7), 0, N * R - 1)
                    didx_v[pl.ds(gi * 16, 16)] = nloc >> 3
                    live = (base2 + gi * 16 + lax.iota(_i32, 16)) < off
                    nv = plsc.load_gather(norm_v, [nloc])
                    val_v[pl.ds(gi * 16, 16)] = jnp.where(live, nv, 0.0)
                plsc.parallel_loop(0, GB // 16, unroll=2)(grp)
                pltpu.async_copy(xw_hbm.at[gidx_v], rows_v, sem).wait()
                def acc(gi, _):
                    dl16 = didx_v[pl.ds(gi * 16, 16)]
                    sv16 = val_v[pl.ds(gi * 16, 16)]
                    for l in range(16):
                        base = dl16[l] * D
                        sv = sv16[l]
                        j = gi * 16 + l
                        for kk in range(D // 16):
                            acc_v[pl.ds(base + kk * 16, 16)] = (
                                acc_v[pl.ds(base + kk * 16, 16)]
                                + rows_v[j, pl.ds(kk * 16, 16)] * sv)
                    return 0
                ngrp = jnp.clip((off - base2 + 15) >> 4, 0, GB // 16)
                lax.fori_loop(0, ngrp, acc, 0)
                return 0
            lax.fori_loop(0, nsub, sub, 0)

        _fire(ew_a, sem, 0)
        _fire(ew_b, sem2, 1)
        def chunk2(c2, _):
            i0 = c2 * 2
            _wait(ew_a, sem, i0)
            _process(ew_a)
            @pl.when(i0 + 2 < NSC3)
            def _():
                _fire(ew_a, sem, i0 + 2)
            _wait(ew_b, sem2, i0 + 1)
            _process(ew_b)
            @pl.when(i0 + 3 < NSC3)
            def _():
                _fire(ew_b, sem2, i0 + 3)
            return 0
        lax.fori_loop(0, NSC3 // 2, chunk2, 0)

        pltpu.sync_copy(
            acc_v,
            agg_hbm.at[pl.ds((c * NHALF + s * TROWS) * D, TROWS * D)])

    return k(xw2, ewf).reshape(NPAD, D)


def _post(agg, h, root, bias, g, b):
    BM = 256
    gm = pl.cdiv(N, BM)

    def body(a_ref, h_ref, r_ref, bias_ref, g_ref, b_ref, o_ref):
        y = a_ref[...] + jnp.dot(h_ref[...].astype(jnp.bfloat16),
                                 r_ref[...].astype(jnp.bfloat16),
                                 preferred_element_type=_f32) + bias_ref[...]
        y = jnp.maximum(y, 0.0)
        mu = jnp.mean(y, axis=-1, keepdims=True)
        var = jnp.mean((y - mu) ** 2, axis=-1, keepdims=True)
        o_ref[...] = (y - mu) * lax.rsqrt(var + EPS) * g_ref[...] + b_ref[...]

    return pl.pallas_call(
        body,
        grid=(gm,),
        in_specs=[pl.BlockSpec((BM, D), lambda i: (i, 0)),
                  pl.BlockSpec((BM, D), lambda i: (i, 0)),
                  pl.BlockSpec((D, D), lambda i: (0, 0)),
                  pl.BlockSpec((1, D), lambda i: (0, 0)),
                  pl.BlockSpec((1, D), lambda i: (0, 0)),
                  pl.BlockSpec((1, D), lambda i: (0, 0))],
        out_specs=pl.BlockSpec((BM, D), lambda i: (i, 0)),
        out_shape=jax.ShapeDtypeStruct((N, D), _f32),
    )(agg, h, root, bias, g, b)


def _layer(h, ew3, basis, comp, root, bias, g, b):
    w3 = _wcat(comp, basis.reshape(NB, D * D)).reshape(R, D, D)
    xw = _xw(h, w3)
    agg = _edge_agg(xw.reshape(N * R, D), ew3)
    return _post(agg[:N], h, root, bias.reshape(1, D), g.reshape(1, D),
                 b.reshape(1, D))


def kernel(x, edge_index, edge_type, emb, basis1, comp1, root1, bias1, g1, b1,
           basis2, comp2, root2, bias2, g2, b2):
    xpad = jnp.concatenate(
        [x.astype(_i32), jnp.zeros((NPAD - N,), _i32)])
    srcp = jnp.concatenate(
        [edge_index[0].astype(_i32), jnp.zeros((EPAD - E,), _i32)])
    keyp = jnp.concatenate(
        [edge_index[1].astype(_i32) * R + edge_type.astype(_i32),
         jnp.full((EPAD - E,), (NPAD - 1) * R, _i32)])
    ew3 = (srcp << 17) | keyp

    h0 = _emb_gather(emb, xpad)[:N]
    h1 = _layer(h0, ew3, basis1, comp1, root1, bias1, g1, b1)
    h2 = _layer(h1, ew3, basis2, comp2, root2, bias2, g2, b2)
    return h2


# speedup vs baseline: 1.0564x; 1.0374x over previous
import functools

import jax
import jax.numpy as jnp
from jax import lax
from jax.experimental import pallas as pl
from jax.experimental.pallas import tpu as pltpu
from jax.experimental.pallas import tpu_sc as plsc

N = 10000
E = 160000
D = 256
R = 8
NB = 30
EPS = 1e-5

NC = 2
NS = 16
NHALF = 5120
NPAD = NC * NHALF
CNTN = NHALF * R
EPAD = 163840
EPT = EPAD // NS
CB = 128
NCHC = EPT // CB
RB = 64
NCHR = EPT // RB
CSL = CNTN // NS
ROWS_T = NHALF // NS
KMASK = 131071

_f32 = jnp.float32
_i32 = jnp.int32


def _emb_gather(emb, xpad):
    bpw = NPAD // (NC * NS)
    mesh = plsc.VectorSubcoreMesh(core_axis_name="c", subcore_axis_name="s")

    @functools.partial(
        pl.kernel, mesh=mesh,
        out_type=jax.ShapeDtypeStruct((NPAD, D), _f32),
        scratch_types=[
            pltpu.VMEM((bpw,), _i32),
            pltpu.VMEM((bpw, D), _f32),
            pltpu.SemaphoreType.DMA,
        ],
    )
    def k(emb_hbm, idx_hbm, out_hbm, idx_v, rows_v, sem):
        wid = lax.axis_index("s") * NC + lax.axis_index("c")
        base = wid * bpw
        pltpu.sync_copy(idx_hbm.at[pl.ds(base, bpw)], idx_v)
        pltpu.async_copy(emb_hbm.at[idx_v], rows_v, sem).wait()
        pltpu.sync_copy(rows_v, out_hbm.at[pl.ds(base, bpw)])

    return k(emb, xpad)


def _wcat(comp, basisf):
    BK = 2048

    def body(c_ref, b_ref, o_ref):
        o_ref[...] = jnp.dot(c_ref[...], b_ref[...],
                             preferred_element_type=_f32)

    return pl.pallas_call(
        body,
        grid=(D * D // BK,),
        in_specs=[pl.BlockSpec((R, NB), lambda j: (0, 0)),
                  pl.BlockSpec((NB, BK), lambda j: (0, j))],
        out_specs=pl.BlockSpec((R, BK), lambda j: (0, j)),
        out_shape=jax.ShapeDtypeStruct((R, D * D), _f32),
    )(comp, basisf)


def _xw(h, w3):
    BM = 256
    gm = pl.cdiv(N, BM)

    def body(h_ref, w_ref, o_ref):
        o_ref[...] = jnp.dot(h_ref[...].astype(jnp.bfloat16),
                             w_ref[0].astype(jnp.bfloat16),
                             preferred_element_type=_f32)

    return pl.pallas_call(
        body,
        grid=(gm, R),
        in_specs=[pl.BlockSpec((BM, D), lambda i, r: (i, 0)),
                  pl.BlockSpec((1, D, D), lambda i, r: (r, 0, 0))],
        out_specs=pl.BlockSpec((BM, D), lambda i, r: (i, r)),
        out_shape=jax.ShapeDtypeStruct((N, R * D), _f32),
    )(h, w3)


TSL = CNTN // NS
TROWS = NHALF // NS
SCB = 2048
NSC1 = EPT // SCB
NSC3 = EPAD // SCB
GB = 64


def _edge_agg(xw2, ewf):
    mesh = plsc.VectorSubcoreMesh(core_axis_name="c", subcore_axis_name="s")

    @functools.partial(
        pl.kernel, mesh=mesh,
        compiler_params=pltpu.CompilerParams(needs_layout_passes=False),
        out_type=jax.ShapeDtypeStruct((NPAD * D,), _f32),
        scratch_types=[
            pltpu.VMEM((SCB,), _i32),
            pltpu.VMEM((SCB,), _i32),
            pltpu.VMEM((SCB + 16,), _i32),
            pltpu.VMEM((NS, CB), _i32),
            pltpu.VMEM((NS, CB), _f32),
            pltpu.VMEM((GB,), _i32),
            pltpu.VMEM((GB,), _i32),
            pltpu.VMEM((GB,), _f32),
            pltpu.VMEM((GB, D), _f32),
            pltpu.VMEM((GB,), _i32),
            pltpu.VMEM((GB,), _i32),
            pltpu.VMEM((GB,), _f32),
            pltpu.VMEM((GB, D), _f32),
            pltpu.VMEM((TSL,), _f32),
            pltpu.VMEM((TROWS * D,), _f32),
            pltpu.VMEM_SHARED((CNTN,), _f32),
            pltpu.SemaphoreType.DMA,
            pltpu.SemaphoreType.DMA,
            pltpu.SemaphoreType.DMA,
            pltpu.SemaphoreType.DMA,
        ],
    )
    def k(xw_hbm, ew_hbm, agg_hbm,
          ew_a, ew_b, cew_v, cidx_v, cval_v, gidx_a, didx_a, val_a, rows_a,
          gidx_b, didx_b, val_b, rows_b, norm_v, acc_v, cnt_sh, sem, sem2,
          sem3, sem4):
        c = lax.axis_index("c")
        s = lax.axis_index("s")
        klo = c * CNTN
        wlo = klo + s * TSL

        def zl(i, _):
            norm_v[pl.ds(i * 16, 16)] = jnp.zeros((16,), _f32)
            return 0
        lax.fori_loop(0, TSL // 16, zl, 0)
        pltpu.sync_copy(norm_v, cnt_sh.at[pl.ds(s * TSL, TSL)])
        def zr(i):
            acc_v[pl.ds(i * 16, 16)] = jnp.zeros((16,), _f32)
        plsc.parallel_loop(0, TROWS * D // 16, unroll=8)(zr)
        plsc.subcore_barrier()

        def cnt_chunk(ci, _):
            pltpu.sync_copy(
                ew_hbm.at[pl.ds(s * EPT + ci * SCB, SCB)], ew_a)
            def sub(si, _):
                def grp(gi):
                    ew = ew_a[pl.ds(si * CB + gi * 16, 16)]
                    key = ew & KMASK
                    member = (key >= klo) & (key < klo + CNTN)
                    lk = jnp.where(key >= CNTN, key - CNTN, key)
                    cidx_v[si, pl.ds(gi * 16, 16)] = lk
                    cval_v[si, pl.ds(gi * 16, 16)] = jnp.where(
                        member, 1.0, 0.0)
                plsc.parallel_loop(0, CB // 16, unroll=4)(grp)
                pltpu.async_copy(cval_v.at[si], cnt_sh.at[cidx_v.at[si]],
                                 sem2, add=True)
                return 0
            lax.fori_loop(0, SCB // CB, sub, 0)
            def drain(si, _):
                pltpu.make_async_copy(cval_v.at[si],
                                      cnt_sh.at[cidx_v.at[si]], sem2).wait()
                return 0
            lax.fori_loop(0, SCB // CB, drain, 0)
            return 0
        lax.fori_loop(0, NSC1, cnt_chunk, 0)
        plsc.subcore_barrier()

        pltpu.sync_copy(cnt_sh.at[pl.ds(s * TSL, TSL)], norm_v)
        def nl(i, _):
            v = norm_v[pl.ds(i * 16, 16)]
            norm_v[pl.ds(i * 16, 16)] = 1.0 / jnp.maximum(v, 1.0)
            return 0
        lax.fori_loop(0, TSL // 16, nl, 0)

        def _fire(slot_ref, slot_sem, idx):
            pltpu.async_copy(ew_hbm.at[pl.ds(idx * SCB, SCB)], slot_ref,
                             slot_sem)

        def _wait(slot_ref, slot_sem, idx):
            pltpu.make_async_copy(ew_hbm.at[pl.ds(idx * SCB, SCB)], slot_ref,
                                  slot_sem).wait()

        def _process(slot_ref):
            def scan(g, off):
                ew = slot_ref[pl.ds(g * 16, 16)]
                key = ew & KMASK
                m = (key >= wlo) & (key < wlo + TSL)
                plsc.store_compressed(cew_v.at[pl.ds(off, 16)], ew, mask=m)
                return off + plsc.all_reduce_population_count(m)[0]
            off = plsc.parallel_loop(0, SCB // 16, unroll=4,
                                     carry=jnp.int32(0))(scan)
            nsub = (off + GB - 1) // GB

            def prep(j, gidx_v, didx_v, val_v):
                base2 = j * GB
                def grp(gi):
                    cw = cew_v[pl.ds(base2 + gi * 16, 16)]
                    key = cw & KMASK
                    nloc = jnp.clip(key - wlo, 0, TSL - 1)
                    gidx_v[pl.ds(gi * 16, 16)] = jnp.clip(
                        ((cw >> 17) << 3) | (key & 7), 0, N * R - 1)
                    didx_v[pl.ds(gi * 16, 16)] = nloc >> 3
                    live = (base2 + gi * 16 + lax.iota(_i32, 16)) < off
                    nv = plsc.load_gather(norm_v, [nloc])
                    val_v[pl.ds(gi * 16, 16)] = jnp.where(live, nv, 0.0)
                plsc.parallel_loop(0, GB // 16, unroll=2)(grp)

            def rmw_sub(j, didx_v, val_v, rows_v):
                def acc(gi, _):
                    dl16 = didx_v[pl.ds(gi * 16, 16)]
                    sv16 = val_v[pl.ds(gi * 16, 16)]
                    for l in range(16):
                        base = dl16[l] * D
                        sv = sv16[l]
                        for kk in range(D // 16):
                            acc_v[pl.ds(base + kk * 16, 16)] = (
                                acc_v[pl.ds(base + kk * 16, 16)]
                                + rows_v[gi * 16 + l, pl.ds(kk * 16, 16)]
                                * sv)
                    return 0
                ngrp = jnp.clip((off - j * GB + 15) >> 4, 0, GB // 16)
                lax.fori_loop(0, ngrp, acc, 0)

            @pl.when(nsub > 0)
            def _():
                prep(0, gidx_a, didx_a, val_a)
                pltpu.async_copy(xw_hbm.at[gidx_a], rows_a, sem3)
                def pair(t, _):
                    j0 = 2 * t
                    j1 = j0 + 1
                    @pl.when(j1 < nsub)
                    def _():
                        prep(j1, gidx_b, didx_b, val_b)
                        pltpu.async_copy(xw_hbm.at[gidx_b], rows_b, sem4)
                    pltpu.make_async_copy(xw_hbm.at[gidx_a], rows_a,
                                          sem3).wait()
                    rmw_sub(j0, didx_a, val_a, rows_a)
                    @pl.when(j1 < nsub)
                    def _():
                        @pl.when(j1 + 1 < nsub)
                        def _():
                            prep(j1 + 1, gidx_a, didx_a, val_a)
                            pltpu.async_copy(xw_hbm.at[gidx_a], rows_a, sem3)
                        pltpu.make_async_copy(xw_hbm.at[gidx_b], rows_b,
                                              sem4).wait()
                        rmw_sub(j1, didx_b, val_b, rows_b)
                    return 0
                lax.fori_loop(0, (nsub + 1) // 2, pair, 0)

        _fire(ew_a, sem, 0)
        _fire(ew_b, sem2, 1)
        def chunk2(c2, _):
            i0 = c2 * 2
            _wait(ew_a, sem, i0)
            _process(ew_a)
            @pl.when(i0 + 2 < NSC3)
            def _():
                _fire(ew_a, sem, i0 + 2)
            _wait(ew_b, sem2, i0 + 1)
            _process(ew_b)
            @pl.when(i0 + 3 < NSC3)
            def _():
                _fire(ew_b, sem2, i0 + 3)
            return 0
        lax.fori_loop(0, NSC3 // 2, chunk2, 0)

        pltpu.sync_copy(
            acc_v,
            agg_hbm.at[pl.ds((c * NHALF + s * TROWS) * D, TROWS * D)])

    return k(xw2, ewf).reshape(NPAD, D)


def _post(agg, h, root, bias, g, b):
    BM = 256
    gm = pl.cdiv(N, BM)

    def body(a_ref, h_ref, r_ref, bias_ref, g_ref, b_ref, o_ref):
        y = a_ref[...] + jnp.dot(h_ref[...].astype(jnp.bfloat16),
                                 r_ref[...].astype(jnp.bfloat16),
                                 preferred_element_type=_f32) + bias_ref[...]
        y = jnp.maximum(y, 0.0)
        mu = jnp.mean(y, axis=-1, keepdims=True)
        var = jnp.mean((y - mu) ** 2, axis=-1, keepdims=True)
        o_ref[...] = (y - mu) * lax.rsqrt(var + EPS) * g_ref[...] + b_ref[...]

    return pl.pallas_call(
        body,
        grid=(gm,),
        in_specs=[pl.BlockSpec((BM, D), lambda i: (i, 0)),
                  pl.BlockSpec((BM, D), lambda i: (i, 0)),
                  pl.BlockSpec((D, D), lambda i: (0, 0)),
                  pl.BlockSpec((1, D), lambda i: (0, 0)),
                  pl.BlockSpec((1, D), lambda i: (0, 0)),
                  pl.BlockSpec((1, D), lambda i: (0, 0))],
        out_specs=pl.BlockSpec((BM, D), lambda i: (i, 0)),
        out_shape=jax.ShapeDtypeStruct((N, D), _f32),
    )(agg, h, root, bias, g, b)


def _layer(h, ew3, basis, comp, root, bias, g, b):
    w3 = _wcat(comp, basis.reshape(NB, D * D)).reshape(R, D, D)
    xw = _xw(h, w3)
    agg = _edge_agg(xw.reshape(N * R, D), ew3)
    return _post(agg[:N], h, root, bias.reshape(1, D), g.reshape(1, D),
                 b.reshape(1, D))


def kernel(x, edge_index, edge_type, emb, basis1, comp1, root1, bias1, g1, b1,
           basis2, comp2, root2, bias2, g2, b2):
    xpad = jnp.concatenate(
        [x.astype(_i32), jnp.zeros((NPAD - N,), _i32)])
    srcp = jnp.concatenate(
        [edge_index[0].astype(_i32), jnp.zeros((EPAD - E,), _i32)])
    keyp = jnp.concatenate(
        [edge_index[1].astype(_i32) * R + edge_type.astype(_i32),
         jnp.full((EPAD - E,), (NPAD - 1) * R, _i32)])
    ew3 = (srcp << 17) | keyp

    h0 = _emb_gather(emb, xpad)[:N]
    h1 = _layer(h0, ew3, basis1, comp1, root1, bias1, g1, b1)
    h2 = _layer(h1, ew3, basis2, comp2, root2, bias2, g2, b2)
    return h2


# speedup vs baseline: 1.6778x; 1.5883x over previous
import functools

import jax
import jax.numpy as jnp
from jax import lax
from jax.experimental import pallas as pl
from jax.experimental.pallas import tpu as pltpu
from jax.experimental.pallas import tpu_sc as plsc

N = 10000
E = 160000
D = 256
R = 8
NB = 30
EPS = 1e-5

NC = 2
NS = 16
NHALF = 5120
NPAD = NC * NHALF
CNTN = NHALF * R
EPAD = 163840
EPT = EPAD // NS
CB = 128
NCHC = EPT // CB
RB = 64
NCHR = EPT // RB
CSL = CNTN // NS
ROWS_T = NHALF // NS
KMASK = 131071

_f32 = jnp.float32
_i32 = jnp.int32


def _emb_gather(emb, xpad):
    bpw = NPAD // (NC * NS)
    mesh = plsc.VectorSubcoreMesh(core_axis_name="c", subcore_axis_name="s")

    @functools.partial(
        pl.kernel, mesh=mesh,
        out_type=jax.ShapeDtypeStruct((NPAD, D), _f32),
        scratch_types=[
            pltpu.VMEM((bpw,), _i32),
            pltpu.VMEM((bpw, D), _f32),
            pltpu.SemaphoreType.DMA,
        ],
    )
    def k(emb_hbm, idx_hbm, out_hbm, idx_v, rows_v, sem):
        wid = lax.axis_index("s") * NC + lax.axis_index("c")
        base = wid * bpw
        pltpu.sync_copy(idx_hbm.at[pl.ds(base, bpw)], idx_v)
        pltpu.async_copy(emb_hbm.at[idx_v], rows_v, sem).wait()
        pltpu.sync_copy(rows_v, out_hbm.at[pl.ds(base, bpw)])

    return k(emb, xpad)


def _wcat(comp, basisf):
    BK = 2048

    def body(c_ref, b_ref, o_ref):
        o_ref[...] = jnp.dot(c_ref[...], b_ref[...],
                             preferred_element_type=_f32)

    return pl.pallas_call(
        body,
        grid=(D * D // BK,),
        in_specs=[pl.BlockSpec((R, NB), lambda j: (0, 0)),
                  pl.BlockSpec((NB, BK), lambda j: (0, j))],
        out_specs=pl.BlockSpec((R, BK), lambda j: (0, j)),
        out_shape=jax.ShapeDtypeStruct((R, D * D), _f32),
    )(comp, basisf)


def _xw(h, w3):
    BM = 256
    gm = pl.cdiv(N, BM)

    def body(h_ref, w_ref, o_ref):
        o_ref[...] = jnp.dot(h_ref[...].astype(jnp.bfloat16),
                             w_ref[0].astype(jnp.bfloat16),
                             preferred_element_type=_f32)

    return pl.pallas_call(
        body,
        grid=(gm, R),
        in_specs=[pl.BlockSpec((BM, D), lambda i, r: (i, 0)),
                  pl.BlockSpec((1, D, D), lambda i, r: (r, 0, 0))],
        out_specs=pl.BlockSpec((BM, D), lambda i, r: (i, r)),
        out_shape=jax.ShapeDtypeStruct((N, R * D), _f32),
    )(h, w3)


TSL = CNTN // NS
TROWS = NHALF // NS
SCB = 4096
CSB = 2048
NSC1 = EPT // CSB
NSC3 = EPAD // SCB
GB = 48


def _edge_agg(xw2, ewf):
    mesh = plsc.VectorSubcoreMesh(core_axis_name="c", subcore_axis_name="s")

    @functools.partial(
        pl.kernel, mesh=mesh,
        compiler_params=pltpu.CompilerParams(needs_layout_passes=False),
        out_type=jax.ShapeDtypeStruct((NPAD * D,), _f32),
        scratch_types=[
            pltpu.VMEM((SCB,), _i32),
            pltpu.VMEM((SCB + 16,), _i32),
            pltpu.VMEM((NS, CB), _i32),
            pltpu.VMEM((NS, CB), _f32),
            pltpu.VMEM((GB,), _i32),
            pltpu.VMEM((GB,), _i32),
            pltpu.VMEM((GB,), _f32),
            pltpu.VMEM((GB, D), _f32),
            pltpu.VMEM((GB,), _i32),
            pltpu.VMEM((GB,), _i32),
            pltpu.VMEM((GB,), _f32),
            pltpu.VMEM((GB, D), _f32),
            pltpu.VMEM((TSL,), _f32),
            pltpu.VMEM((TROWS * D,), _f32),
            pltpu.VMEM_SHARED((CNTN,), _f32),
            pltpu.SemaphoreType.DMA,
            pltpu.SemaphoreType.DMA,
            pltpu.SemaphoreType.DMA,
            pltpu.SemaphoreType.DMA,
        ],
    )
    def k(xw_hbm, ew_hbm, agg_hbm,
          ew_a, cew_v, cidx_v, cval_v, gidx_a, didx_a, val_a, rows_a,
          gidx_b, didx_b, val_b, rows_b, norm_v, acc_v, cnt_sh, sem, sem2,
          sem3, sem4):
        c = lax.axis_index("c")
        s = lax.axis_index("s")
        klo = c * CNTN
        wlo = klo + s * TSL

        def zl(i, _):
            norm_v[pl.ds(i * 16, 16)] = jnp.zeros((16,), _f32)
            return 0
        lax.fori_loop(0, TSL // 16, zl, 0)
        pltpu.sync_copy(norm_v, cnt_sh.at[pl.ds(s * TSL, TSL)])
        def zr(i):
            acc_v[pl.ds(i * 16, 16)] = jnp.zeros((16,), _f32)
        plsc.parallel_loop(0, TROWS * D // 16, unroll=8)(zr)
        plsc.subcore_barrier()

        def cnt_chunk(ci, _):
            pltpu.sync_copy(
                ew_hbm.at[pl.ds(s * EPT + ci * CSB, CSB)], ew_a.at[pl.ds(0, CSB)])
            def sub(si, _):
                def grp(gi):
                    ew = ew_a[pl.ds(si * CB + gi * 16, 16)]
                    key = ew & KMASK
                    member = (key >= klo) & (key < klo + CNTN)
                    lk = jnp.where(key >= CNTN, key - CNTN, key)
                    cidx_v[si, pl.ds(gi * 16, 16)] = lk
                    cval_v[si, pl.ds(gi * 16, 16)] = jnp.where(
                        member, 1.0, 0.0)
                plsc.parallel_loop(0, CB // 16, unroll=4)(grp)
                pltpu.async_copy(cval_v.at[si], cnt_sh.at[cidx_v.at[si]],
                                 sem2, add=True)
                return 0
            lax.fori_loop(0, CSB // CB, sub, 0)
            def drain(si, _):
                pltpu.make_async_copy(cval_v.at[si],
                                      cnt_sh.at[cidx_v.at[si]], sem2).wait()
                return 0
            lax.fori_loop(0, CSB // CB, drain, 0)
            return 0
        lax.fori_loop(0, NSC1, cnt_chunk, 0)
        plsc.subcore_barrier()

        pltpu.sync_copy(cnt_sh.at[pl.ds(s * TSL, TSL)], norm_v)
        def nl(i, _):
            v = norm_v[pl.ds(i * 16, 16)]
            norm_v[pl.ds(i * 16, 16)] = 1.0 / jnp.maximum(v, 1.0)
            return 0
        lax.fori_loop(0, TSL // 16, nl, 0)

        def _process(slot_ref):
            def scan(g, off):
                ew = slot_ref[pl.ds(g * 16, 16)]
                key = ew & KMASK
                m = (key >= wlo) & (key < wlo + TSL)
                plsc.store_compressed(cew_v.at[pl.ds(off, 16)], ew, mask=m)
                return off + plsc.all_reduce_population_count(m)[0]
            off = plsc.parallel_loop(0, SCB // 16, unroll=4,
                                     carry=jnp.int32(0))(scan)
            nsub = (off + GB - 1) // GB

            def prep(j, gidx_v, didx_v, val_v):
                base2 = j * GB
                def grp(gi):
                    cw = cew_v[pl.ds(base2 + gi * 16, 16)]
                    key = cw & KMASK
                    nloc = jnp.clip(key - wlo, 0, TSL - 1)
                    gidx_v[pl.ds(gi * 16, 16)] = jnp.clip(
                        ((cw >> 17) << 3) | (key & 7), 0, N * R - 1)
                    didx_v[pl.ds(gi * 16, 16)] = nloc >> 3
                    live = (base2 + gi * 16 + lax.iota(_i32, 16)) < off
                    nv = plsc.load_gather(norm_v, [nloc])
                    val_v[pl.ds(gi * 16, 16)] = jnp.where(live, nv, 0.0)
                plsc.parallel_loop(0, GB // 16, unroll=2)(grp)

            def rmw_sub(j, didx_v, val_v, rows_v):
                def acc(gi, _):
                    dl16 = didx_v[pl.ds(gi * 16, 16)]
                    sv16 = val_v[pl.ds(gi * 16, 16)]
                    for l in range(16):
                        base = dl16[l] * D
                        sv = sv16[l]
                        for kk in range(D // 16):
                            acc_v[pl.ds(base + kk * 16, 16)] = (
                                acc_v[pl.ds(base + kk * 16, 16)]
                                + rows_v[gi * 16 + l, pl.ds(kk * 16, 16)]
                                * sv)
                    return 0
                ngrp = jnp.clip((off - j * GB + 15) >> 4, 0, GB // 16)
                lax.fori_loop(0, ngrp, acc, 0)

            @pl.when(nsub > 0)
            def _():
                prep(0, gidx_a, didx_a, val_a)
                pltpu.async_copy(xw_hbm.at[gidx_a], rows_a, sem3)
                def pair(t, _):
                    j0 = 2 * t
                    j1 = j0 + 1
                    @pl.when(j1 < nsub)
                    def _():
                        prep(j1, gidx_b, didx_b, val_b)
                        pltpu.async_copy(xw_hbm.at[gidx_b], rows_b, sem4)
                    pltpu.make_async_copy(xw_hbm.at[gidx_a], rows_a,
                                          sem3).wait()
                    rmw_sub(j0, didx_a, val_a, rows_a)
                    @pl.when(j1 < nsub)
                    def _():
                        @pl.when(j1 + 1 < nsub)
                        def _():
                            prep(j1 + 1, gidx_a, didx_a, val_a)
                            pltpu.async_copy(xw_hbm.at[gidx_a], rows_a, sem3)
                        pltpu.make_async_copy(xw_hbm.at[gidx_b], rows_b,
                                              sem4).wait()
                        rmw_sub(j1, didx_b, val_b, rows_b)
                    return 0
                lax.fori_loop(0, (nsub + 1) // 2, pair, 0)

        def chunk1(c2, _):
            pltpu.sync_copy(ew_hbm.at[pl.ds(c2 * SCB, SCB)], ew_a)
            _process(ew_a)
            return 0
        lax.fori_loop(0, NSC3, chunk1, 0)

        pltpu.sync_copy(
            acc_v,
            agg_hbm.at[pl.ds((c * NHALF + s * TROWS) * D, TROWS * D)])

    return k(xw2, ewf).reshape(NPAD, D)


def _post(agg, h, root, bias, g, b):
    BM = 256
    gm = pl.cdiv(N, BM)

    def body(a_ref, h_ref, r_ref, bias_ref, g_ref, b_ref, o_ref):
        y = a_ref[...] + jnp.dot(h_ref[...].astype(jnp.bfloat16),
                                 r_ref[...].astype(jnp.bfloat16),
                                 preferred_element_type=_f32) + bias_ref[...]
        y = jnp.maximum(y, 0.0)
        mu = jnp.mean(y, axis=-1, keepdims=True)
        var = jnp.mean((y - mu) ** 2, axis=-1, keepdims=True)
        o_ref[...] = (y - mu) * lax.rsqrt(var + EPS) * g_ref[...] + b_ref[...]

    return pl.pallas_call(
        body,
        grid=(gm,),
        in_specs=[pl.BlockSpec((BM, D), lambda i: (i, 0)),
                  pl.BlockSpec((BM, D), lambda i: (i, 0)),
                  pl.BlockSpec((D, D), lambda i: (0, 0)),
                  pl.BlockSpec((1, D), lambda i: (0, 0)),
                  pl.BlockSpec((1, D), lambda i: (0, 0)),
                  pl.BlockSpec((1, D), lambda i: (0, 0))],
        out_specs=pl.BlockSpec((BM, D), lambda i: (i, 0)),
        out_shape=jax.ShapeDtypeStruct((N, D), _f32),
    )(agg, h, root, bias, g, b)


def _layer(h, ew3, basis, comp, root, bias, g, b):
    w3 = _wcat(comp, basis.reshape(NB, D * D)).reshape(R, D, D)
    xw = _xw(h, w3)
    agg = _edge_agg(xw.reshape(N * R, D), ew3)
    return _post(agg[:N], h, root, bias.reshape(1, D), g.reshape(1, D),
                 b.reshape(1, D))


def kernel(x, edge_index, edge_type, emb, basis1, comp1, root1, bias1, g1, b1,
           basis2, comp2, root2, bias2, g2, b2):
    xpad = jnp.concatenate(
        [x.astype(_i32), jnp.zeros((NPAD - N,), _i32)])
    srcp = jnp.concatenate(
        [edge_index[0].astype(_i32), jnp.zeros((EPAD - E,), _i32)])
    keyp = jnp.concatenate(
        [edge_index[1].astype(_i32) * R + edge_type.astype(_i32),
         jnp.full((EPAD - E,), (NPAD - 1) * R, _i32)])
    ew3 = (srcp << 17) | keyp

    h0 = _emb_gather(emb, xpad)[:N]
    h1 = _layer(h0, ew3, basis1, comp1, root1, bias1, g1, b1)
    h2 = _layer(h1, ew3, basis2, comp2, root2, bias2, g2, b2)
    return h2
